# baseline (speedup 1.0000x reference)
import functools

import jax
import jax.numpy as jnp
from jax import lax
from jax.experimental import pallas as pl
from jax.experimental.pallas import tpu as pltpu
from jax.experimental.pallas import tpu_sc as plsc

_NC = 2
_NS = 16
_EK = 128
_BN = 1000


def _npad(n_nodes):
    align = 8 * _NS
    return ((n_nodes + align - 1) // align) * align


def _sc_mesh():
    return plsc.VectorSubcoreMesh(core_axis_name="c", subcore_axis_name="s",
                                  num_cores=_NC, num_subcores=_NS)


def _build_sc_agg(n_nodes, ept_pad, n_chunks, with_cnt=False):
    npad = _npad(n_nodes)
    rows_t = npad // _NS
    rounds = ept_pad // _EK
    cpc = n_chunks // _NC
    agg_t = jax.ShapeDtypeStruct((n_chunks * npad, 128), jnp.float32)
    cnt_t = jax.ShapeDtypeStruct((_NC * npad, 128), jnp.float32)
    out_type = (agg_t, cnt_t) if with_cnt else agg_t
    scratch = (
        pltpu.VMEM_SHARED((npad, 128), jnp.float32),
        pltpu.VMEM((ept_pad // 2,), jnp.int32),
        pltpu.VMEM((rounds, _EK), jnp.int32),
        pltpu.VMEM((_EK,), jnp.int32),
        pltpu.VMEM((_EK,), jnp.int32),
        pltpu.VMEM((_EK, 128), jnp.float32),
        pltpu.VMEM((_EK, 128), jnp.float32),
        pltpu.SemaphoreType.DMA,
        pltpu.SemaphoreType.DMA,
        pltpu.SemaphoreType.DMA,
        pltpu.SemaphoreType.DMA,
    )

    def body(*refs):
        if with_cnt:
            (x_h, src_h, dst_h, z_h, on_h, agg_o, cnt_o,
             acc_s, gpk, didx, stg0, stg1, gb0, gb1,
             sem0, sem1, sso0, sso1) = refs
        else:
            (x_h, src_h, dst_h, z_h, agg_o,
             acc_s, gpk, didx, stg0, stg1, gb0, gb1,
             sem0, sem1, sso0, sso1) = refs
        c = lax.axis_index("c")
        t = lax.axis_index("s")
        row0 = t * rows_t
        pltpu.sync_copy(src_h.at[t], gpk)
        pltpu.sync_copy(dst_h.at[t], didx)

        if with_cnt:
            pltpu.sync_copy(on_h, gb0)
            pltpu.sync_copy(z_h, acc_s.at[pl.ds(row0, rows_t)])
            plsc.subcore_barrier()

            def cbody(r, carry):
                @pl.when(lax.rem(r, 2) == c)
                def _():
                    pltpu.sync_copy(gb0, acc_s.at[didx.at[r]], add=True)
                return carry

            lax.fori_loop(0, rounds, cbody, 0)
            plsc.subcore_barrier()
            pltpu.sync_copy(acc_s.at[pl.ds(row0, rows_t)],
                            cnt_o.at[pl.ds(c * npad + row0, rows_t)])
            plsc.subcore_barrier()

        for ci in range(cpc):
            chunk = c + _NC * ci

            def stage(r, stg, chunk=chunk):
                for w in range(_EK // 32):
                    v = gpk[pl.ds(r * (_EK // 2) + w * 16, 16)]
                    lo = jnp.bitwise_and(v, 0xFFFF)
                    hi = lax.shift_right_logical(v, 16)
                    stg[pl.ds(w * 32, 16)] = lo * n_chunks + chunk
                    stg[pl.ds(w * 32 + 16, 16)] = hi * n_chunks + chunk

            pltpu.sync_copy(z_h, acc_s.at[pl.ds(row0, rows_t)])
            plsc.subcore_barrier()

            stage(0, stg0)
            pltpu.async_copy(x_h.at[stg0], gb0, sem0)

            def rbody(r, carry):
                def do(stgA, gbA, semA, ssoA, stgB, gbB, semB, ssoB):
                    pltpu.make_async_copy(x_h.at[stgA], gbA, semA).wait()
                    pltpu.async_copy(gbA, acc_s.at[didx.at[r]], ssoA, add=True)

                    @pl.when(r + 1 < rounds)
                    def _():
                        @pl.when(r >= 1)
                        def _():
                            pltpu.make_async_copy(
                                gbB, acc_s.at[didx.at[0]], ssoB).wait()

                        stage(r + 1, stgB)
                        pltpu.async_copy(x_h.at[stgB], gbB, semB)

                @pl.when(lax.rem(r, 2) == 0)
                def _():
                    do(stg0, gb0, sem0, sso0, stg1, gb1, sem1, sso1)

                @pl.when(lax.rem(r, 2) == 1)
                def _():
                    do(stg1, gb1, sem1, sso1, stg0, gb0, sem0, sso0)

                return carry

            lax.fori_loop(0, rounds, rbody, 0)
            pltpu.make_async_copy(gb0, acc_s.at[didx.at[0]], sso0).wait()
            pltpu.make_async_copy(gb1, acc_s.at[didx.at[0]], sso1).wait()
            plsc.subcore_barrier()

            out0 = chunk * npad + row0
            pltpu.sync_copy(acc_s.at[pl.ds(row0, rows_t)],
                            agg_o.at[pl.ds(out0, rows_t)])
            plsc.subcore_barrier()

    return pl.kernel(body, out_type=out_type, mesh=_sc_mesh(),
                     scratch_types=scratch)


def _tc_sage1(agg, cnt, x, w_cat, b_row):
    n = x.shape[0]
    grid = n // _BN
    d = x.shape[1]
    h = w_cat.shape[1]

    def body(aref, cref, xref, wref, bref, oref):
        cs = cref[0] + cref[1]
        inv = 1.0 / jnp.maximum(cs[:, 0:1], 1.0)
        xa = jnp.concatenate([aref[0] * inv, aref[1] * inv, xref[...]], axis=1)
        acc = jnp.dot(xa, wref[...], preferred_element_type=jnp.float32)
        oref[...] = jnp.maximum(acc + bref[...], 0.0)

    return pl.pallas_call(
        body,
        grid=(grid,),
        in_specs=[
            pl.BlockSpec((2, _BN, 128), lambda i: (0, i, 0)),
            pl.BlockSpec((2, _BN, 128), lambda i: (0, i, 0)),
            pl.BlockSpec((_BN, d), lambda i: (i, 0)),
            pl.BlockSpec((2 * d, h), lambda i: (0, 0)),
            pl.BlockSpec((1, h), lambda i: (0, 0)),
        ],
        out_specs=pl.BlockSpec((_BN, h), lambda i: (i, 0)),
        out_shape=jax.ShapeDtypeStruct((n, h), jnp.float32),
        compiler_params=pltpu.CompilerParams(dimension_semantics=("arbitrary",)),
    )(agg, cnt, x, w_cat, b_row)


def _tc_sage2_pool_head(agg, cnt, hin, batch3, meta_p, w_cat, b_row,
                        wh1g, wh1m, bh1r, wh2, bh2r, wh3, bh3r):
    n = hin.shape[0]
    grid = n // _BN
    h = hin.shape[1]
    g = meta_p.shape[0]
    mp = meta_p.shape[1]
    k1 = wh1g.shape[1]
    k2 = wh2.shape[1]
    k3 = wh3.shape[1]

    def body(aref, cref, href, bref, mref, wref, b2ref, wgref, wmref, b1ref,
             w2ref, bt2ref, w3ref, bt3ref, oref, yref, pool, gc):
        i = pl.program_id(0)
        cs = cref[0] + cref[1]
        inv = 1.0 / jnp.maximum(cs[:, 0:1], 1.0)
        xa = jnp.concatenate([aref[0] * inv, aref[1] * inv,
                              aref[2] * inv, aref[3] * inv, href[...]], axis=1)
        hh = jnp.maximum(
            jnp.dot(xa, wref[...], preferred_element_type=jnp.float32)
            + b2ref[...], 0.0)
        oref[...] = hh

        bb = bref[0]
        oh = (lax.broadcasted_iota(jnp.int32, (g, _BN), 0) == bb
              ).astype(jnp.float32)

        @pl.when(i == 0)
        def _():
            pool[...] = jnp.zeros_like(pool)
            gc[...] = jnp.zeros_like(gc)

        pool[...] += jnp.dot(oh, hh, preferred_element_type=jnp.float32)
        gc[...] += jnp.broadcast_to(jnp.sum(oh, axis=1, keepdims=True), gc.shape)

        @pl.when(i == grid - 1)
        def _():
            gmean = pool[...] / jnp.maximum(gc[...][:, 0:1], 1.0)
            t1 = jnp.maximum(
                jnp.dot(gmean, wgref[...], preferred_element_type=jnp.float32)
                + jnp.dot(mref[...], wmref[...], preferred_element_type=jnp.float32)
                + b1ref[...], 0.0)
            t2 = jnp.maximum(
                jnp.dot(t1, w2ref[...], preferred_element_type=jnp.float32)
                + bt2ref[...], 0.0)
            yref[...] = (jnp.dot(t2, w3ref[...], preferred_element_type=jnp.float32)
                         + bt3ref[...])

    return pl.pallas_call(
        body,
        grid=(grid,),
        in_specs=[
            pl.BlockSpec((4, _BN, 128), lambda i: (0, i, 0)),
            pl.BlockSpec((2, _BN, 128), lambda i: (0, i, 0)),
            pl.BlockSpec((_BN, h), lambda i: (i, 0)),
            pl.BlockSpec((1, 1, _BN), lambda i: (i, 0, 0)),
            pl.BlockSpec((g, mp), lambda i: (0, 0)),
            pl.BlockSpec((2 * h, h), lambda i: (0, 0)),
            pl.BlockSpec((1, h), lambda i: (0, 0)),
            pl.BlockSpec((h, k1), lambda i: (0, 0)),
            pl.BlockSpec((mp, k1), lambda i: (0, 0)),
            pl.BlockSpec((1, k1), lambda i: (0, 0)),
            pl.BlockSpec((k1, k2), lambda i: (0, 0)),
            pl.BlockSpec((1, k2), lambda i: (0, 0)),
            pl.BlockSpec((k2, k3), lambda i: (0, 0)),
            pl.BlockSpec((1, k3), lambda i: (0, 0)),
        ],
        out_specs=[
            pl.BlockSpec((_BN, h), lambda i: (i, 0)),
            pl.BlockSpec((g, k3), lambda i: (0, 0)),
        ],
        out_shape=[
            jax.ShapeDtypeStruct((n, h), jnp.float32),
            jax.ShapeDtypeStruct((g, k3), jnp.float32),
        ],
        scratch_shapes=[
            pltpu.VMEM((g, h), jnp.float32),
            pltpu.VMEM((g, 128), jnp.float32),
        ],
        compiler_params=pltpu.CompilerParams(dimension_semantics=("arbitrary",)),
    )(agg, cnt, hin, batch3, meta_p, w_cat, b_row,
      wh1g, wh1m, bh1r, wh2, bh2r, wh3, bh3r)


def kernel(x, edge_index, batch, meta, Wl1, Wr1, b1, Wl2, Wr2, b2,
           Wh1, bh1, Wh2, bh2, Wh3, bh3):
    n, d = x.shape
    e = edge_index.shape[1]
    h = Wl1.shape[1]
    g, m = meta.shape
    npad = _npad(n)

    ept_pad = -(-(e // _NS) // _EK) * _EK
    pad = _NS * ept_pad - e
    src_f = jnp.concatenate([edge_index[0], jnp.zeros((pad,), jnp.int32)])
    trash = n + jnp.arange(pad, dtype=jnp.int32) % (npad - n)
    dst_f = jnp.concatenate([edge_index[1], trash])
    s5 = src_f.reshape(_NS, ept_pad // 32, 2, 16)
    src_pk = (s5[:, :, 0, :] | (s5[:, :, 1, :] << 16)).reshape(_NS, ept_pad // 2)
    dst3 = dst_f.reshape(_NS, ept_pad // _EK, _EK)
    zrows = jnp.zeros((npad // _NS, 128), jnp.float32)
    ones = jnp.ones((_EK, 128), jnp.float32)

    xflat = x.reshape(n * (d // 128), 128)
    agg1f, cntf = _build_sc_agg(n, ept_pad, d // 128, with_cnt=True)(
        xflat, src_pk, dst3, zrows, ones)
    agg1 = agg1f.reshape(d // 128, npad, 128)
    cnt = cntf.reshape(2, npad, 128)
    w1cat = jnp.concatenate([Wl1, Wr1], axis=0)
    h1 = _tc_sage1(agg1, cnt, x, w1cat, b1.reshape(1, h))

    hflat = h1.reshape(n * (h // 128), 128)
    agg2 = _build_sc_agg(n, ept_pad, h // 128)(
        hflat, src_pk, dst3, zrows).reshape(h // 128, npad, 128)
    w2cat = jnp.concatenate([Wl2, Wr2], axis=0)
    batch3 = batch.reshape(n // _BN, 1, _BN)
    meta_p = jnp.pad(meta, ((0, 0), (0, 8 - m)))
    wh1g = Wh1[:h]
    wh1m = jnp.pad(Wh1[h:], ((0, 8 - m), (0, 0)))
    node_emb, yhat = _tc_sage2_pool_head(
        agg2, cnt, h1, batch3, meta_p, w2cat, b2.reshape(1, h),
        wh1g, wh1m, bh1.reshape(1, -1), Wh2, bh2.reshape(1, -1),
        Wh3, bh3.reshape(1, -1))
    return yhat, node_emb


# speedup vs baseline: 1.3489x; 1.3489x over previous
import functools

import jax
import jax.numpy as jnp
from jax import lax
from jax.experimental import pallas as pl
from jax.experimental.pallas import tpu as pltpu
from jax.experimental.pallas import tpu_sc as plsc

_NC = 2
_NS = 16
_EK = 80
_BN = 1000


def _npad(n_nodes):
    align = 8 * _NS
    return ((n_nodes + align - 1) // align) * align


def _sc_mesh():
    return plsc.VectorSubcoreMesh(core_axis_name="c", subcore_axis_name="s",
                                  num_cores=_NC, num_subcores=_NS)


def _build_sc_agg(n_nodes, ept_pad, n_chunks, with_cnt=False):
    npad = _npad(n_nodes)
    rows_t = npad // _NS
    rounds = ept_pad // _EK
    cpc = n_chunks // _NC
    agg_t = jax.ShapeDtypeStruct((n_chunks * npad, 128), jnp.float32)
    cnt_t = jax.ShapeDtypeStruct((_NC * npad, 128), jnp.float32)
    out_type = (agg_t, cnt_t) if with_cnt else agg_t
    scratch = (
        pltpu.VMEM_SHARED((npad, 128), jnp.float32),
        pltpu.VMEM((ept_pad,), jnp.int32),
        pltpu.VMEM((rounds, _EK), jnp.int32),
        pltpu.VMEM((_EK, 128), jnp.float32),
        pltpu.VMEM((_EK, 128), jnp.float32),
        pltpu.SemaphoreType.DMA,
        pltpu.SemaphoreType.DMA,
        pltpu.SemaphoreType.DMA,
        pltpu.SemaphoreType.DMA,
    )

    def body(*refs):
        if with_cnt:
            (x_h, src_h, dst_h, z_h, on_h, agg_o, cnt_o,
             acc_s, gidx, didx, gb0, gb1,
             sem0, sem1, sso0, sso1) = refs
        else:
            (x_h, src_h, dst_h, z_h, agg_o,
             acc_s, gidx, didx, gb0, gb1,
             sem0, sem1, sso0, sso1) = refs
        c = lax.axis_index("c")
        t = lax.axis_index("s")
        row0 = t * rows_t
        pltpu.sync_copy(src_h.at[t], gidx)
        pltpu.sync_copy(dst_h.at[t], didx)

        if with_cnt:
            pltpu.sync_copy(on_h, gb0)
            pltpu.sync_copy(z_h, acc_s.at[pl.ds(row0, rows_t)])
            plsc.subcore_barrier()

            def cbody(r, carry):
                @pl.when(lax.rem(r, 2) == c)
                def _():
                    pltpu.sync_copy(gb0, acc_s.at[didx.at[r]], add=True)
                return carry

            lax.fori_loop(0, rounds, cbody, 0)
            plsc.subcore_barrier()
            pltpu.sync_copy(acc_s.at[pl.ds(row0, rows_t)],
                            cnt_o.at[pl.ds(c * npad + row0, rows_t)])
            plsc.subcore_barrier()

        for ci in range(cpc):
            chunk = c + _NC * ci

            def xform(j, carry, ci=ci, chunk=chunk):
                sl = pl.ds(j * 16, 16)
                if ci == 0:
                    gidx[sl] = gidx[sl] * n_chunks + chunk
                else:
                    gidx[sl] = gidx[sl] + _NC
                return carry

            lax.fori_loop(0, ept_pad // 16, xform, 0)

            pltpu.sync_copy(z_h, acc_s.at[pl.ds(row0, rows_t)])
            plsc.subcore_barrier()

            pltpu.async_copy(x_h.at[gidx.at[pl.ds(0, _EK)]], gb0, sem0)

            def rbody(r, carry):
                def do(gbA, semA, ssoA, gbB, semB, ssoB):
                    pltpu.make_async_copy(
                        x_h.at[gidx.at[pl.ds(r * _EK, _EK)]], gbA, semA).wait()
                    pltpu.async_copy(gbA, acc_s.at[didx.at[r]], ssoA, add=True)

                    @pl.when(r + 1 < rounds)
                    def _():
                        @pl.when(r >= 1)
                        def _():
                            pltpu.make_async_copy(
                                gbB, acc_s.at[didx.at[0]], ssoB).wait()

                        pltpu.async_copy(
                            x_h.at[gidx.at[pl.ds((r + 1) * _EK, _EK)]], gbB, semB)

                @pl.when(lax.rem(r, 2) == 0)
                def _():
                    do(gb0, sem0, sso0, gb1, sem1, sso1)

                @pl.when(lax.rem(r, 2) == 1)
                def _():
                    do(gb1, sem1, sso1, gb0, sem0, sso0)

                return carry

            lax.fori_loop(0, rounds, rbody, 0)
            pltpu.make_async_copy(gb0, acc_s.at[didx.at[0]], sso0).wait()
            pltpu.make_async_copy(gb1, acc_s.at[didx.at[0]], sso1).wait()
            plsc.subcore_barrier()

            out0 = chunk * npad + row0
            pltpu.sync_copy(acc_s.at[pl.ds(row0, rows_t)],
                            agg_o.at[pl.ds(out0, rows_t)])
            plsc.subcore_barrier()

    return pl.kernel(body, out_type=out_type, mesh=_sc_mesh(),
                     scratch_types=scratch)


def _tc_sage1(agg, cnt, x, w_cat, b_row):
    n = x.shape[0]
    grid = n // _BN
    d = x.shape[1]
    h = w_cat.shape[1]

    def body(aref, cref, xref, wref, bref, oref):
        cs = cref[0] + cref[1]
        inv = 1.0 / jnp.maximum(cs[:, 0:1], 1.0)
        xa = jnp.concatenate([aref[0] * inv, aref[1] * inv, xref[...]], axis=1)
        acc = jnp.dot(xa, wref[...], preferred_element_type=jnp.float32)
        oref[...] = jnp.maximum(acc + bref[...], 0.0)

    return pl.pallas_call(
        body,
        grid=(grid,),
        in_specs=[
            pl.BlockSpec((2, _BN, 128), lambda i: (0, i, 0)),
            pl.BlockSpec((2, _BN, 128), lambda i: (0, i, 0)),
            pl.BlockSpec((_BN, d), lambda i: (i, 0)),
            pl.BlockSpec((2 * d, h), lambda i: (0, 0)),
            pl.BlockSpec((1, h), lambda i: (0, 0)),
        ],
        out_specs=pl.BlockSpec((_BN, h), lambda i: (i, 0)),
        out_shape=jax.ShapeDtypeStruct((n, h), jnp.float32),
        compiler_params=pltpu.CompilerParams(dimension_semantics=("arbitrary",)),
    )(agg, cnt, x, w_cat, b_row)


def _tc_sage2_pool_head(agg, cnt, hin, batch3, meta_p, w_cat, b_row,
                        wh1g, wh1m, bh1r, wh2, bh2r, wh3, bh3r):
    n = hin.shape[0]
    grid = n // _BN
    h = hin.shape[1]
    g = meta_p.shape[0]
    mp = meta_p.shape[1]
    k1 = wh1g.shape[1]
    k2 = wh2.shape[1]
    k3 = wh3.shape[1]

    def body(aref, cref, href, bref, mref, wref, b2ref, wgref, wmref, b1ref,
             w2ref, bt2ref, w3ref, bt3ref, oref, yref, pool, gc):
        i = pl.program_id(0)
        cs = cref[0] + cref[1]
        inv = 1.0 / jnp.maximum(cs[:, 0:1], 1.0)
        xa = jnp.concatenate([aref[0] * inv, aref[1] * inv,
                              aref[2] * inv, aref[3] * inv, href[...]], axis=1)
        hh = jnp.maximum(
            jnp.dot(xa, wref[...], preferred_element_type=jnp.float32)
            + b2ref[...], 0.0)
        oref[...] = hh

        bb = bref[0]
        oh = (lax.broadcasted_iota(jnp.int32, (g, _BN), 0) == bb
              ).astype(jnp.float32)

        @pl.when(i == 0)
        def _():
            pool[...] = jnp.zeros_like(pool)
            gc[...] = jnp.zeros_like(gc)

        pool[...] += jnp.dot(oh, hh, preferred_element_type=jnp.float32)
        gc[...] += jnp.broadcast_to(jnp.sum(oh, axis=1, keepdims=True), gc.shape)

        @pl.when(i == grid - 1)
        def _():
            gmean = pool[...] / jnp.maximum(gc[...][:, 0:1], 1.0)
            t1 = jnp.maximum(
                jnp.dot(gmean, wgref[...], preferred_element_type=jnp.float32)
                + jnp.dot(mref[...], wmref[...], preferred_element_type=jnp.float32)
                + b1ref[...], 0.0)
            t2 = jnp.maximum(
                jnp.dot(t1, w2ref[...], preferred_element_type=jnp.float32)
                + bt2ref[...], 0.0)
            yref[...] = (jnp.dot(t2, w3ref[...], preferred_element_type=jnp.float32)
                         + bt3ref[...])

    return pl.pallas_call(
        body,
        grid=(grid,),
        in_specs=[
            pl.BlockSpec((4, _BN, 128), lambda i: (0, i, 0)),
            pl.BlockSpec((2, _BN, 128), lambda i: (0, i, 0)),
            pl.BlockSpec((_BN, h), lambda i: (i, 0)),
            pl.BlockSpec((1, 1, _BN), lambda i: (i, 0, 0)),
            pl.BlockSpec((g, mp), lambda i: (0, 0)),
            pl.BlockSpec((2 * h, h), lambda i: (0, 0)),
            pl.BlockSpec((1, h), lambda i: (0, 0)),
            pl.BlockSpec((h, k1), lambda i: (0, 0)),
            pl.BlockSpec((mp, k1), lambda i: (0, 0)),
            pl.BlockSpec((1, k1), lambda i: (0, 0)),
            pl.BlockSpec((k1, k2), lambda i: (0, 0)),
            pl.BlockSpec((1, k2), lambda i: (0, 0)),
            pl.BlockSpec((k2, k3), lambda i: (0, 0)),
            pl.BlockSpec((1, k3), lambda i: (0, 0)),
        ],
        out_specs=[
            pl.BlockSpec((_BN, h), lambda i: (i, 0)),
            pl.BlockSpec((g, k3), lambda i: (0, 0)),
        ],
        out_shape=[
            jax.ShapeDtypeStruct((n, h), jnp.float32),
            jax.ShapeDtypeStruct((g, k3), jnp.float32),
        ],
        scratch_shapes=[
            pltpu.VMEM((g, h), jnp.float32),
            pltpu.VMEM((g, 128), jnp.float32),
        ],
        compiler_params=pltpu.CompilerParams(dimension_semantics=("arbitrary",)),
    )(agg, cnt, hin, batch3, meta_p, w_cat, b_row,
      wh1g, wh1m, bh1r, wh2, bh2r, wh3, bh3r)


def kernel(x, edge_index, batch, meta, Wl1, Wr1, b1, Wl2, Wr2, b2,
           Wh1, bh1, Wh2, bh2, Wh3, bh3):
    n, d = x.shape
    e = edge_index.shape[1]
    h = Wl1.shape[1]
    g, m = meta.shape
    npad = _npad(n)

    ept_pad = e // _NS
    src_pk = edge_index[0].reshape(_NS, ept_pad)
    dst3 = edge_index[1].reshape(_NS, ept_pad // _EK, _EK)
    zrows = jnp.zeros((npad // _NS, 128), jnp.float32)
    ones = jnp.ones((_EK, 128), jnp.float32)

    xflat = x.reshape(n * (d // 128), 128)
    agg1f, cntf = _build_sc_agg(n, ept_pad, d // 128, with_cnt=True)(
        xflat, src_pk, dst3, zrows, ones)
    agg1 = agg1f.reshape(d // 128, npad, 128)
    cnt = cntf.reshape(2, npad, 128)
    w1cat = jnp.concatenate([Wl1, Wr1], axis=0)
    h1 = _tc_sage1(agg1, cnt, x, w1cat, b1.reshape(1, h))

    hflat = h1.reshape(n * (h // 128), 128)
    agg2 = _build_sc_agg(n, ept_pad, h // 128)(
        hflat, src_pk, dst3, zrows).reshape(h // 128, npad, 128)
    w2cat = jnp.concatenate([Wl2, Wr2], axis=0)
    batch3 = batch.reshape(n // _BN, 1, _BN)
    meta_p = jnp.pad(meta, ((0, 0), (0, 8 - m)))
    wh1g = Wh1[:h]
    wh1m = jnp.pad(Wh1[h:], ((0, 8 - m), (0, 0)))
    node_emb, yhat = _tc_sage2_pool_head(
        agg2, cnt, h1, batch3, meta_p, w2cat, b2.reshape(1, h),
        wh1g, wh1m, bh1.reshape(1, -1), Wh2, bh2.reshape(1, -1),
        Wh3, bh3.reshape(1, -1))
    return yhat, node_emb


# speedup vs baseline: 1.3505x; 1.0011x over previous
import functools

import jax
import jax.numpy as jnp
from jax import lax
from jax.experimental import pallas as pl
from jax.experimental.pallas import tpu as pltpu
from jax.experimental.pallas import tpu_sc as plsc

_NC = 2
_NS = 16
_EK = 80
_BN = 1000


def _npad(n_nodes):
    align = 8 * _NS
    return ((n_nodes + align - 1) // align) * align


def _sc_mesh():
    return plsc.VectorSubcoreMesh(core_axis_name="c", subcore_axis_name="s",
                                  num_cores=_NC, num_subcores=_NS)


def _build_sc_agg(n_nodes, ept_pad, n_chunks, with_cnt=False):
    npad = _npad(n_nodes)
    rows_t = npad // _NS
    rounds = ept_pad // _EK
    cpc = n_chunks // _NC
    agg_t = jax.ShapeDtypeStruct((n_chunks * npad, 128), jnp.float32)
    cnt_t = jax.ShapeDtypeStruct((_NC * npad, 128), jnp.float32)
    out_type = (agg_t, cnt_t) if with_cnt else agg_t
    scratch = (
        pltpu.VMEM_SHARED((npad, 128), jnp.float32),
        pltpu.VMEM((ept_pad,), jnp.int32),
        pltpu.VMEM((rounds, _EK), jnp.int32),
        pltpu.VMEM((_EK, 128), jnp.float32),
        pltpu.VMEM((_EK, 128), jnp.float32),
        pltpu.SemaphoreType.DMA,
        pltpu.SemaphoreType.DMA,
        pltpu.SemaphoreType.DMA,
        pltpu.SemaphoreType.DMA,
    )

    def body(*refs):
        if with_cnt:
            (x_h, src_h, dst_h, z_h, on_h, agg_o, cnt_o,
             acc_s, gidx, didx, gb0, gb1,
             sem0, sem1, sso0, sso1) = refs
        else:
            (x_h, src_h, dst_h, z_h, agg_o,
             acc_s, gidx, didx, gb0, gb1,
             sem0, sem1, sso0, sso1) = refs
        c = lax.axis_index("c")
        t = lax.axis_index("s")
        row0 = t * rows_t
        pltpu.sync_copy(src_h.at[t], gidx)
        pltpu.sync_copy(dst_h.at[t], didx)

        if with_cnt:
            pltpu.sync_copy(on_h, gb0)
            pltpu.sync_copy(z_h, acc_s.at[pl.ds(row0, rows_t)])
            plsc.subcore_barrier()

            def cbody(r, carry):
                @pl.when(lax.rem(r, 2) == c)
                def _():
                    pltpu.sync_copy(gb0, acc_s.at[didx.at[r]], add=True)
                return carry

            lax.fori_loop(0, rounds, cbody, 0)
            plsc.subcore_barrier()
            pltpu.sync_copy(acc_s.at[pl.ds(row0, rows_t)],
                            cnt_o.at[pl.ds(c * npad + row0, rows_t)])
            plsc.subcore_barrier()

        for ci in range(cpc):
            chunk = c + _NC * ci

            def xform(j, carry, ci=ci, chunk=chunk):
                sl = pl.ds(j * 16, 16)
                if ci == 0:
                    gidx[sl] = gidx[sl] * n_chunks + chunk
                else:
                    gidx[sl] = gidx[sl] + _NC
                return carry

            lax.fori_loop(0, ept_pad // 16, xform, 0)

            pltpu.sync_copy(z_h, acc_s.at[pl.ds(row0, rows_t)])
            plsc.subcore_barrier()

            pltpu.async_copy(x_h.at[gidx.at[pl.ds(0, _EK)]], gb0, sem0)

            def rbody(r, carry):
                def do(gbA, semA, ssoA, gbB, semB, ssoB):
                    pltpu.make_async_copy(
                        x_h.at[gidx.at[pl.ds(r * _EK, _EK)]], gbA, semA).wait()
                    pltpu.async_copy(gbA, acc_s.at[didx.at[r]], ssoA, add=True)

                    @pl.when(r + 1 < rounds)
                    def _():
                        @pl.when(r >= 1)
                        def _():
                            pltpu.make_async_copy(
                                gbB, acc_s.at[didx.at[0]], ssoB).wait()

                        pltpu.async_copy(
                            x_h.at[gidx.at[pl.ds((r + 1) * _EK, _EK)]], gbB, semB)

                @pl.when(lax.rem(r, 2) == 0)
                def _():
                    do(gb0, sem0, sso0, gb1, sem1, sso1)

                @pl.when(lax.rem(r, 2) == 1)
                def _():
                    do(gb1, sem1, sso1, gb0, sem0, sso0)

                return carry

            lax.fori_loop(0, rounds, rbody, 0)
            pltpu.make_async_copy(gb0, acc_s.at[didx.at[0]], sso0).wait()
            pltpu.make_async_copy(gb1, acc_s.at[didx.at[0]], sso1).wait()
            plsc.subcore_barrier()

            out0 = chunk * npad + row0
            pltpu.sync_copy(acc_s.at[pl.ds(row0, rows_t)],
                            agg_o.at[pl.ds(out0, rows_t)])
            plsc.subcore_barrier()

    return pl.kernel(body, out_type=out_type, mesh=_sc_mesh(),
                     scratch_types=scratch)


def _tc_sage1(agg, cnt, x, w_cat, b_row):
    n = x.shape[0]
    grid = n // _BN
    d = x.shape[1]
    h = w_cat.shape[1]

    def body(aref, cref, xref, wref, bref, oref):
        cs = cref[0] + cref[1]
        inv = 1.0 / jnp.maximum(cs[:, 0:1], 1.0)
        xa = jnp.concatenate([aref[0] * inv, aref[1] * inv, xref[...]], axis=1)
        acc = jnp.dot(xa.astype(jnp.bfloat16), wref[...].astype(jnp.bfloat16),
                      preferred_element_type=jnp.float32)
        oref[...] = jnp.maximum(acc + bref[...], 0.0)

    return pl.pallas_call(
        body,
        grid=(grid,),
        in_specs=[
            pl.BlockSpec((2, _BN, 128), lambda i: (0, i, 0)),
            pl.BlockSpec((2, _BN, 128), lambda i: (0, i, 0)),
            pl.BlockSpec((_BN, d), lambda i: (i, 0)),
            pl.BlockSpec((2 * d, h), lambda i: (0, 0)),
            pl.BlockSpec((1, h), lambda i: (0, 0)),
        ],
        out_specs=pl.BlockSpec((_BN, h), lambda i: (i, 0)),
        out_shape=jax.ShapeDtypeStruct((n, h), jnp.float32),
        compiler_params=pltpu.CompilerParams(dimension_semantics=("arbitrary",)),
    )(agg, cnt, x, w_cat, b_row)


def _tc_sage2_pool_head(agg, cnt, hin, batch3, meta_p, w_cat, b_row,
                        wh1g, wh1m, bh1r, wh2, bh2r, wh3, bh3r):
    n = hin.shape[0]
    grid = n // _BN
    h = hin.shape[1]
    g = meta_p.shape[0]
    mp = meta_p.shape[1]
    k1 = wh1g.shape[1]
    k2 = wh2.shape[1]
    k3 = wh3.shape[1]

    def body(aref, cref, href, bref, mref, wref, b2ref, wgref, wmref, b1ref,
             w2ref, bt2ref, w3ref, bt3ref, oref, yref, pool, gc):
        i = pl.program_id(0)
        cs = cref[0] + cref[1]
        inv = 1.0 / jnp.maximum(cs[:, 0:1], 1.0)
        xa = jnp.concatenate([aref[0] * inv, aref[1] * inv,
                              aref[2] * inv, aref[3] * inv, href[...]], axis=1)
        hh = jnp.maximum(
            jnp.dot(xa.astype(jnp.bfloat16), wref[...].astype(jnp.bfloat16),
                    preferred_element_type=jnp.float32)
            + b2ref[...], 0.0)
        oref[...] = hh

        bb = bref[0]
        oh = (lax.broadcasted_iota(jnp.int32, (g, _BN), 0) == bb
              ).astype(jnp.float32)

        @pl.when(i == 0)
        def _():
            pool[...] = jnp.zeros_like(pool)
            gc[...] = jnp.zeros_like(gc)

        pool[...] += jnp.dot(oh, hh, preferred_element_type=jnp.float32)
        gc[...] += jnp.broadcast_to(jnp.sum(oh, axis=1, keepdims=True), gc.shape)

        @pl.when(i == grid - 1)
        def _():
            gmean = pool[...] / jnp.maximum(gc[...][:, 0:1], 1.0)
            t1 = jnp.maximum(
                jnp.dot(gmean, wgref[...], preferred_element_type=jnp.float32)
                + jnp.dot(mref[...], wmref[...], preferred_element_type=jnp.float32)
                + b1ref[...], 0.0)
            t2 = jnp.maximum(
                jnp.dot(t1, w2ref[...], preferred_element_type=jnp.float32)
                + bt2ref[...], 0.0)
            yref[...] = (jnp.dot(t2, w3ref[...], preferred_element_type=jnp.float32)
                         + bt3ref[...])

    return pl.pallas_call(
        body,
        grid=(grid,),
        in_specs=[
            pl.BlockSpec((4, _BN, 128), lambda i: (0, i, 0)),
            pl.BlockSpec((2, _BN, 128), lambda i: (0, i, 0)),
            pl.BlockSpec((_BN, h), lambda i: (i, 0)),
            pl.BlockSpec((1, 1, _BN), lambda i: (i, 0, 0)),
            pl.BlockSpec((g, mp), lambda i: (0, 0)),
            pl.BlockSpec((2 * h, h), lambda i: (0, 0)),
            pl.BlockSpec((1, h), lambda i: (0, 0)),
            pl.BlockSpec((h, k1), lambda i: (0, 0)),
            pl.BlockSpec((mp, k1), lambda i: (0, 0)),
            pl.BlockSpec((1, k1), lambda i: (0, 0)),
            pl.BlockSpec((k1, k2), lambda i: (0, 0)),
            pl.BlockSpec((1, k2), lambda i: (0, 0)),
            pl.BlockSpec((k2, k3), lambda i: (0, 0)),
            pl.BlockSpec((1, k3), lambda i: (0, 0)),
        ],
        out_specs=[
            pl.BlockSpec((_BN, h), lambda i: (i, 0)),
            pl.BlockSpec((g, k3), lambda i: (0, 0)),
        ],
        out_shape=[
            jax.ShapeDtypeStruct((n, h), jnp.float32),
            jax.ShapeDtypeStruct((g, k3), jnp.float32),
        ],
        scratch_shapes=[
            pltpu.VMEM((g, h), jnp.float32),
            pltpu.VMEM((g, 128), jnp.float32),
        ],
        compiler_params=pltpu.CompilerParams(dimension_semantics=("arbitrary",)),
    )(agg, cnt, hin, batch3, meta_p, w_cat, b_row,
      wh1g, wh1m, bh1r, wh2, bh2r, wh3, bh3r)


def kernel(x, edge_index, batch, meta, Wl1, Wr1, b1, Wl2, Wr2, b2,
           Wh1, bh1, Wh2, bh2, Wh3, bh3):
    n, d = x.shape
    e = edge_index.shape[1]
    h = Wl1.shape[1]
    g, m = meta.shape
    npad = _npad(n)

    ept_pad = e // _NS
    src_pk = edge_index[0].reshape(_NS, ept_pad)
    dst3 = edge_index[1].reshape(_NS, ept_pad // _EK, _EK)
    zrows = jnp.zeros((npad // _NS, 128), jnp.float32)
    ones = jnp.ones((_EK, 128), jnp.float32)

    xflat = x.reshape(n * (d // 128), 128)
    agg1f, cntf = _build_sc_agg(n, ept_pad, d // 128, with_cnt=True)(
        xflat, src_pk, dst3, zrows, ones)
    agg1 = agg1f.reshape(d // 128, npad, 128)
    cnt = cntf.reshape(2, npad, 128)
    w1cat = jnp.concatenate([Wl1, Wr1], axis=0)
    h1 = _tc_sage1(agg1, cnt, x, w1cat, b1.reshape(1, h))

    hflat = h1.reshape(n * (h // 128), 128)
    agg2 = _build_sc_agg(n, ept_pad, h // 128)(
        hflat, src_pk, dst3, zrows).reshape(h // 128, npad, 128)
    w2cat = jnp.concatenate([Wl2, Wr2], axis=0)
    batch3 = batch.reshape(n // _BN, 1, _BN)
    meta_p = jnp.pad(meta, ((0, 0), (0, 8 - m)))
    wh1g = Wh1[:h]
    wh1m = jnp.pad(Wh1[h:], ((0, 8 - m), (0, 0)))
    node_emb, yhat = _tc_sage2_pool_head(
        agg2, cnt, h1, batch3, meta_p, w2cat, b2.reshape(1, h),
        wh1g, wh1m, bh1.reshape(1, -1), Wh2, bh2.reshape(1, -1),
        Wh3, bh3.reshape(1, -1))
    return yhat, node_emb
